# baseline (speedup 1.0000x reference)
import jax
import jax.numpy as jnp
from jax.experimental import pallas as pl
from jax.experimental.pallas import tpu as pltpu

_F = 128
_OUT = 64
_N = 10000
_N2 = _N // 2


def _body(x_ref, wz_ref, wh_ref, wl_ref, bxz_ref, bhz_ref, bxh_ref, bhh_ref,
          bl_ref, out_ref):
    wz = wz_ref[:]
    wh = wh_ref[:]
    wl = wl_ref[:]
    bz = bxz_ref[0] + bhz_ref[0]
    bh = bxh_ref[0] + bhh_ref[0]
    bl = bl_ref[0]

    def half(xs):
        az = jnp.dot(xs, wz, preferred_element_type=jnp.float32)
        ah = jnp.dot(xs, wh, preferred_element_type=jnp.float32)
        z = jax.nn.sigmoid(az + bz)
        t = jnp.tanh(ah + bh)
        h = jnp.maximum((1.0 - z) * t, 0.0)
        return jnp.dot(h, wl, preferred_element_type=jnp.float32) + bl

    xb = x_ref[:]
    out_ref[:, 0:_OUT] = half(xb[:, 0:_F])
    out_ref[:, _OUT:2 * _OUT] = half(xb[:, _F:2 * _F])


def kernel(x, edge_index, edge_weight, W_xz, b_xz, W_hz, b_hz, W_xr, b_xr,
           W_hr, b_hr, W_xh, b_xh, W_hh, b_hh, W_lin, b_lin):
    del edge_index, edge_weight, W_hz, W_xr, b_xr, W_hr, b_hr, W_hh

    out2 = pl.pallas_call(
        _body,
        out_shape=jax.ShapeDtypeStruct((_N2, 2 * _OUT), jnp.float32),
    )(x.reshape(_N2, 2 * _F), W_xz, W_xh, W_lin, b_xz.reshape(1, _F),
      b_hz.reshape(1, _F), b_xh.reshape(1, _F), b_hh.reshape(1, _F),
      b_lin.reshape(1, _OUT))
    return (out2.reshape(_N, _OUT),)


# speedup vs baseline: 1.3069x; 1.3069x over previous
import jax
import jax.numpy as jnp
from jax.experimental import pallas as pl
from jax.experimental.pallas import tpu as pltpu

_F = 128
_OUT = 64
_N = 10000
_C = 1000
_NC = _N // _C
_IS = 3
_OS = 4


def _body(x_hbm, wz_ref, wh_ref, wl_ref, bxz_ref, bhz_ref, bxh_ref, bhh_ref,
          bl_ref, out_hbm, xbuf, obuf, in_sem, out_sem):
    def copy_in(slot, idx):
        return pltpu.make_async_copy(
            x_hbm.at[pl.ds(idx * _C, _C), :], xbuf.at[slot], in_sem.at[slot])

    def copy_out(slot, idx):
        return pltpu.make_async_copy(
            obuf.at[slot], out_hbm.at[pl.ds(idx * _C, _C), :], out_sem.at[slot])

    for j in range(_IS - 1):
        copy_in(j, j).start()

    bz = bxz_ref[0] + bhz_ref[0]
    bh = bxh_ref[0] + bhh_ref[0]
    bl = bl_ref[0]
    wz = wz_ref[:]
    wh = wh_ref[:]
    wl = wl_ref[:]

    for i in range(_NC):
        if i + _IS - 1 < _NC:
            copy_in((i + _IS - 1) % _IS, i + _IS - 1).start()
        copy_in(i % _IS, i).wait()
        xb = xbuf[i % _IS]
        az = jnp.dot(xb, wz, preferred_element_type=jnp.float32)
        ah = jnp.dot(xb, wh, preferred_element_type=jnp.float32)
        z = jax.nn.sigmoid(az + bz)
        t = jnp.tanh(ah + bh)
        h = jnp.maximum((1.0 - z) * t, 0.0)
        if i >= _OS:
            copy_out(i % _OS, i - _OS).wait()
        obuf[i % _OS] = jnp.dot(h, wl, preferred_element_type=jnp.float32) + bl
        copy_out(i % _OS, i).start()
    for i in range(max(0, _NC - _OS), _NC):
        copy_out(i % _OS, i).wait()


def kernel(x, edge_index, edge_weight, W_xz, b_xz, W_hz, b_hz, W_xr, b_xr,
           W_hr, b_hr, W_xh, b_xh, W_hh, b_hh, W_lin, b_lin):
    del edge_index, edge_weight, W_hz, W_xr, b_xr, W_hr, b_hr, W_hh

    vmem = pl.BlockSpec(memory_space=pltpu.MemorySpace.VMEM)
    hbm = pl.BlockSpec(memory_space=pltpu.MemorySpace.HBM)
    out = pl.pallas_call(
        _body,
        in_specs=[hbm, vmem, vmem, vmem, vmem, vmem, vmem, vmem, vmem],
        out_specs=hbm,
        out_shape=jax.ShapeDtypeStruct((_N, _OUT), jnp.float32),
        scratch_shapes=[
            pltpu.VMEM((_IS, _C, _F), jnp.float32),
            pltpu.VMEM((_OS, _C, _OUT), jnp.float32),
            pltpu.SemaphoreType.DMA((_IS,)),
            pltpu.SemaphoreType.DMA((_OS,)),
        ],
    )(x, W_xz, W_xh, W_lin, b_xz.reshape(1, _F), b_hz.reshape(1, _F),
      b_xh.reshape(1, _F), b_hh.reshape(1, _F), b_lin.reshape(1, _OUT))
    return (out,)


# speedup vs baseline: 1.6474x; 1.2605x over previous
import jax
import jax.numpy as jnp
from jax.experimental import pallas as pl
from jax.experimental.pallas import tpu as pltpu

_F = 128
_OUT = 64
_N = 10000
_C = 2000
_NC = _N // _C


def _body(x_hbm, wz_ref, wh_ref, wl_ref, bxz_ref, bhz_ref, bxh_ref, bhh_ref,
          bl_ref, out_hbm, *bufs):
    xbufs = bufs[0:_NC]
    obufs = bufs[_NC:2 * _NC]
    in_sems = bufs[2 * _NC:3 * _NC]
    out_sems = bufs[3 * _NC:4 * _NC]

    def copy_in(idx):
        return pltpu.make_async_copy(
            x_hbm.at[pl.ds(idx * _C, _C), :], xbufs[idx], in_sems[idx])

    def copy_out(idx):
        return pltpu.make_async_copy(
            obufs[idx], out_hbm.at[pl.ds(idx * _C, _C), :], out_sems[idx])

    for i in range(_NC):
        copy_in(i).start()

    bz = bxz_ref[0] + bhz_ref[0]
    bh = bxh_ref[0] + bhh_ref[0]
    bl = bl_ref[0]
    wz = wz_ref[:]
    wh = wh_ref[:]
    wl = wl_ref[:]

    for i in range(_NC):
        copy_in(i).wait()
        xb = xbufs[i][:]
        az = jnp.dot(xb, wz, preferred_element_type=jnp.float32)
        ah = jnp.dot(xb, wh, preferred_element_type=jnp.float32)
        z = jax.nn.sigmoid(az + bz)
        t = jnp.tanh(ah + bh)
        h = jnp.maximum((1.0 - z) * t, 0.0)
        obufs[i][:] = jnp.dot(h, wl, preferred_element_type=jnp.float32) + bl
        copy_out(i).start()
    for i in range(_NC):
        copy_out(i).wait()


def kernel(x, edge_index, edge_weight, W_xz, b_xz, W_hz, b_hz, W_xr, b_xr,
           W_hr, b_hr, W_xh, b_xh, W_hh, b_hh, W_lin, b_lin):
    del edge_index, edge_weight, W_hz, W_xr, b_xr, W_hr, b_hr, W_hh

    vmem = pl.BlockSpec(memory_space=pltpu.MemorySpace.VMEM)
    hbm = pl.BlockSpec(memory_space=pltpu.MemorySpace.HBM)
    scratch = (
        [pltpu.VMEM((_C, _F), jnp.float32) for _ in range(_NC)]
        + [pltpu.VMEM((_C, _OUT), jnp.float32) for _ in range(_NC)]
        + [pltpu.SemaphoreType.DMA for _ in range(2 * _NC)]
    )
    out = pl.pallas_call(
        _body,
        in_specs=[hbm, vmem, vmem, vmem, vmem, vmem, vmem, vmem, vmem],
        out_specs=hbm,
        out_shape=jax.ShapeDtypeStruct((_N, _OUT), jnp.float32),
        scratch_shapes=scratch,
    )(x, W_xz, W_xh, W_lin, b_xz.reshape(1, _F), b_hz.reshape(1, _F),
      b_xh.reshape(1, _F), b_hh.reshape(1, _F), b_lin.reshape(1, _OUT))
    return (out,)
